# baseline (speedup 1.0000x reference)
import jax
import jax.numpy as jnp
from jax.experimental import pallas as pl
from jax.experimental.pallas import tpu as pltpu

NUM_GROUPS = 2
TOKENS_PER_GROUP = 4096
HIDDEN_DIM = 4096
NUM_EXPERTS = 64

BLOCK_T = 512
DEPTH = 4
TOTAL = NUM_GROUPS * TOKENS_PER_GROUP
NCH = TOTAL // BLOCK_T


def _router_ring(x_hbm, w_ref, b_ref, probs_hbm, logits_hbm, zacc_ref,
                 xbuf, pbuf, lbuf, insem, psem, lsem):
    def in_copy(c, slot):
        return pltpu.make_async_copy(
            x_hbm.at[pl.ds(c * BLOCK_T, BLOCK_T), :], xbuf.at[slot],
            insem.at[slot])

    for s in range(DEPTH):
        in_copy(s, s).start()

    def step(i, zsum):
        slot = jax.lax.rem(i, DEPTH)
        in_copy(i, slot).wait()
        x = xbuf[slot]
        logits = jax.lax.dot_general(
            x, w_ref[...], (((1,), (1,)), ((), ())),
            preferred_element_type=jnp.float32,
        ) + b_ref[...]
        m = jnp.max(logits, axis=-1, keepdims=True)
        e = jnp.exp(logits - m)
        s_ = jnp.sum(e, axis=-1, keepdims=True)
        lbuf[i] = logits
        pbuf[i] = e / s_
        pltpu.make_async_copy(
            lbuf.at[i], logits_hbm.at[pl.ds(i * BLOCK_T, BLOCK_T), :],
            lsem.at[i]).start()
        pltpu.make_async_copy(
            pbuf.at[i], probs_hbm.at[pl.ds(i * BLOCK_T, BLOCK_T), :],
            psem.at[i]).start()

        @pl.when(i + DEPTH < NCH)
        def _next():
            in_copy(i + DEPTH, slot).start()

        log_z = m + jnp.log(s_)
        return zsum + jnp.sum(log_z * log_z)

    zsum = jax.lax.fori_loop(0, NCH, step, jnp.float32(0.0))
    zacc_ref[...] = (zsum / TOTAL).reshape(1, 1)

    def drain(i, carry):
        pltpu.make_async_copy(
            lbuf.at[i], logits_hbm.at[pl.ds(i * BLOCK_T, BLOCK_T), :],
            lsem.at[i]).wait()
        pltpu.make_async_copy(
            pbuf.at[i], probs_hbm.at[pl.ds(i * BLOCK_T, BLOCK_T), :],
            psem.at[i]).wait()
        return carry

    jax.lax.fori_loop(0, NCH, drain, 0)


def kernel(token_inputs, W, b, expert_capacity):
    del expert_capacity
    x = token_inputs.reshape(TOTAL, HIDDEN_DIM)
    b2 = b.reshape(1, NUM_EXPERTS)

    probs, logits, zloss = pl.pallas_call(
        _router_ring,
        in_specs=[
            pl.BlockSpec(memory_space=pltpu.MemorySpace.HBM),
            pl.BlockSpec(memory_space=pltpu.MemorySpace.VMEM),
            pl.BlockSpec(memory_space=pltpu.MemorySpace.VMEM),
        ],
        out_specs=[
            pl.BlockSpec(memory_space=pltpu.MemorySpace.HBM),
            pl.BlockSpec(memory_space=pltpu.MemorySpace.HBM),
            pl.BlockSpec(memory_space=pltpu.MemorySpace.VMEM),
        ],
        out_shape=[
            jax.ShapeDtypeStruct((TOTAL, NUM_EXPERTS), jnp.float32),
            jax.ShapeDtypeStruct((TOTAL, NUM_EXPERTS), jnp.float32),
            jax.ShapeDtypeStruct((1, 1), jnp.float32),
        ],
        scratch_shapes=[
            pltpu.VMEM((DEPTH, BLOCK_T, HIDDEN_DIM), jnp.float32),
            pltpu.VMEM((NCH, BLOCK_T, NUM_EXPERTS), jnp.float32),
            pltpu.VMEM((NCH, BLOCK_T, NUM_EXPERTS), jnp.float32),
            pltpu.SemaphoreType.DMA((DEPTH,)),
            pltpu.SemaphoreType.DMA((NCH,)),
            pltpu.SemaphoreType.DMA((NCH,)),
        ],
        compiler_params=pltpu.CompilerParams(
            vmem_limit_bytes=100 * 1024 * 1024,
        ),
    )(x, W, b2)

    router_probs = probs.reshape(NUM_GROUPS, TOKENS_PER_GROUP, NUM_EXPERTS)
    router_logits = logits.reshape(NUM_GROUPS, TOKENS_PER_GROUP, NUM_EXPERTS)
    return (router_probs, router_logits, zloss.reshape(()))


# speedup vs baseline: 1.0187x; 1.0187x over previous
import jax
import jax.numpy as jnp
from jax.experimental import pallas as pl
from jax.experimental.pallas import tpu as pltpu

NUM_GROUPS = 2
TOKENS_PER_GROUP = 4096
HIDDEN_DIM = 4096
NUM_EXPERTS = 64

BLOCK_T = 256
DEPTH = 8
TOTAL = NUM_GROUPS * TOKENS_PER_GROUP
NCH = TOTAL // BLOCK_T


def _router_ring(x_hbm, w_ref, b_ref, probs_hbm, logits_hbm, zacc_ref,
                 xbuf, pbuf, lbuf, zvec, insem, psem, lsem):
    def in_copy(c, slot):
        return pltpu.make_async_copy(
            x_hbm.at[pl.ds(c * BLOCK_T, BLOCK_T), :], xbuf.at[slot],
            insem.at[slot])

    for s in range(DEPTH):
        in_copy(s, s).start()

    zvec[...] = jnp.zeros((BLOCK_T, 1), jnp.float32)

    def step(i, carry):
        slot = jax.lax.rem(i, DEPTH)
        in_copy(i, slot).wait()
        x = xbuf[slot]
        logits = jax.lax.dot_general(
            x, w_ref[...], (((1,), (1,)), ((), ())),
            preferred_element_type=jnp.float32,
        ) + b_ref[...]
        m = jnp.max(logits, axis=-1, keepdims=True)
        e = jnp.exp(logits - m)
        s_ = jnp.sum(e, axis=-1, keepdims=True)
        lbuf[i] = logits
        pbuf[i] = e / s_
        pltpu.make_async_copy(
            lbuf.at[i], logits_hbm.at[pl.ds(i * BLOCK_T, BLOCK_T), :],
            lsem.at[i]).start()
        pltpu.make_async_copy(
            pbuf.at[i], probs_hbm.at[pl.ds(i * BLOCK_T, BLOCK_T), :],
            psem.at[i]).start()

        @pl.when(i + DEPTH < NCH)
        def _next():
            in_copy(i + DEPTH, slot).start()

        log_z = m + jnp.log(s_)
        zvec[...] += log_z * log_z
        return carry

    jax.lax.fori_loop(0, NCH, step, 0)
    zacc_ref[...] = (jnp.sum(zvec[...]) / TOTAL).reshape(1, 1)

    def drain(i, carry):
        pltpu.make_async_copy(
            lbuf.at[i], logits_hbm.at[pl.ds(i * BLOCK_T, BLOCK_T), :],
            lsem.at[i]).wait()
        pltpu.make_async_copy(
            pbuf.at[i], probs_hbm.at[pl.ds(i * BLOCK_T, BLOCK_T), :],
            psem.at[i]).wait()
        return carry

    jax.lax.fori_loop(0, NCH, drain, 0)


def kernel(token_inputs, W, b, expert_capacity):
    del expert_capacity
    x = token_inputs.reshape(TOTAL, HIDDEN_DIM)
    b2 = b.reshape(1, NUM_EXPERTS)

    probs, logits, zloss = pl.pallas_call(
        _router_ring,
        in_specs=[
            pl.BlockSpec(memory_space=pltpu.MemorySpace.HBM),
            pl.BlockSpec(memory_space=pltpu.MemorySpace.VMEM),
            pl.BlockSpec(memory_space=pltpu.MemorySpace.VMEM),
        ],
        out_specs=[
            pl.BlockSpec(memory_space=pltpu.MemorySpace.HBM),
            pl.BlockSpec(memory_space=pltpu.MemorySpace.HBM),
            pl.BlockSpec(memory_space=pltpu.MemorySpace.VMEM),
        ],
        out_shape=[
            jax.ShapeDtypeStruct((TOTAL, NUM_EXPERTS), jnp.float32),
            jax.ShapeDtypeStruct((TOTAL, NUM_EXPERTS), jnp.float32),
            jax.ShapeDtypeStruct((1, 1), jnp.float32),
        ],
        scratch_shapes=[
            pltpu.VMEM((DEPTH, BLOCK_T, HIDDEN_DIM), jnp.float32),
            pltpu.VMEM((NCH, BLOCK_T, NUM_EXPERTS), jnp.float32),
            pltpu.VMEM((NCH, BLOCK_T, NUM_EXPERTS), jnp.float32),
            pltpu.VMEM((BLOCK_T, 1), jnp.float32),
            pltpu.SemaphoreType.DMA((DEPTH,)),
            pltpu.SemaphoreType.DMA((NCH,)),
            pltpu.SemaphoreType.DMA((NCH,)),
        ],
        compiler_params=pltpu.CompilerParams(
            vmem_limit_bytes=100 * 1024 * 1024,
        ),
    )(x, W, b2)

    router_probs = probs.reshape(NUM_GROUPS, TOKENS_PER_GROUP, NUM_EXPERTS)
    router_logits = logits.reshape(NUM_GROUPS, TOKENS_PER_GROUP, NUM_EXPERTS)
    return (router_probs, router_logits, zloss.reshape(()))


# speedup vs baseline: 1.0363x; 1.0173x over previous
import jax
import jax.numpy as jnp
from jax.experimental import pallas as pl

NUM_GROUPS = 2
TOKENS_PER_GROUP = 4096
HIDDEN_DIM = 4096
NUM_EXPERTS = 64

BLOCK_T = 1024


def _router_block(x_ref, w_ref, b_ref, probs_ref, logits_ref, zacc_ref):
    g = pl.program_id(0)
    i = pl.program_id(1)
    x = x_ref[0]
    w = w_ref[...]
    logits = jax.lax.dot_general(
        x, w, (((1,), (1,)), ((), ())), preferred_element_type=jnp.float32
    ) + b_ref[...]
    m = jnp.max(logits, axis=-1, keepdims=True)
    e = jnp.exp(logits - m)
    s = jnp.sum(e, axis=-1, keepdims=True)
    logits_ref[0] = logits
    probs_ref[0] = e / s
    log_z = m + jnp.log(s)
    partial = jnp.sum(log_z * log_z).reshape(1, 1)

    @pl.when((g == 0) & (i == 0))
    def _init():
        zacc_ref[...] = jnp.zeros((1, 1), jnp.float32)

    zacc_ref[...] += partial

    last = (g == NUM_GROUPS - 1) & (i == pl.num_programs(1) - 1)

    @pl.when(last)
    def _finalize():
        zacc_ref[...] *= 1.0 / (NUM_GROUPS * TOKENS_PER_GROUP)


def kernel(token_inputs, W, b, expert_capacity):
    del expert_capacity
    n_blocks = TOKENS_PER_GROUP // BLOCK_T
    b2 = b.reshape(1, NUM_EXPERTS)

    probs, logits, zloss = pl.pallas_call(
        _router_block,
        grid=(NUM_GROUPS, n_blocks),
        in_specs=[
            pl.BlockSpec(
                (1, BLOCK_T, HIDDEN_DIM),
                lambda g, i: (g, i, 0),

            ),
            pl.BlockSpec((NUM_EXPERTS, HIDDEN_DIM), lambda g, i: (0, 0)),
            pl.BlockSpec((1, NUM_EXPERTS), lambda g, i: (0, 0)),
        ],
        out_specs=[
            pl.BlockSpec((1, BLOCK_T, NUM_EXPERTS), lambda g, i: (g, i, 0)),
            pl.BlockSpec((1, BLOCK_T, NUM_EXPERTS), lambda g, i: (g, i, 0)),
            pl.BlockSpec((1, 1), lambda g, i: (0, 0)),
        ],
        out_shape=[
            jax.ShapeDtypeStruct((NUM_GROUPS, TOKENS_PER_GROUP, NUM_EXPERTS), jnp.float32),
            jax.ShapeDtypeStruct((NUM_GROUPS, TOKENS_PER_GROUP, NUM_EXPERTS), jnp.float32),
            jax.ShapeDtypeStruct((1, 1), jnp.float32),
        ],
    )(token_inputs, W, b2)

    return (probs, logits, zloss.reshape(()))


# speedup vs baseline: 1.0364x; 1.0000x over previous
import jax
import jax.numpy as jnp
from jax.experimental import pallas as pl

NUM_GROUPS = 2
TOKENS_PER_GROUP = 4096
HIDDEN_DIM = 4096
NUM_EXPERTS = 64

BLOCK_T = 1024
TOTAL = NUM_GROUPS * TOKENS_PER_GROUP


def _router_block(x_ref, w_ref, b_ref, probs_ref, logits_ref, zacc_ref):
    i = pl.program_id(0)
    x = x_ref[...]
    w = w_ref[...]
    logits = jax.lax.dot_general(
        x, w, (((1,), (1,)), ((), ())), preferred_element_type=jnp.float32
    ) + b_ref[...]
    m = jnp.max(logits, axis=-1, keepdims=True)
    e = jnp.exp(logits - m)
    s = jnp.sum(e, axis=-1, keepdims=True)
    logits_ref[...] = logits
    probs_ref[...] = e / s
    log_z = m + jnp.log(s)
    partial = jnp.sum(log_z * log_z).reshape(1, 1)

    @pl.when(i == 0)
    def _init():
        zacc_ref[...] = jnp.zeros((1, 1), jnp.float32)

    zacc_ref[...] += partial

    @pl.when(i == pl.num_programs(0) - 1)
    def _finalize():
        zacc_ref[...] *= 1.0 / TOTAL


def kernel(token_inputs, W, b, expert_capacity):
    del expert_capacity
    x = token_inputs.reshape(TOTAL, HIDDEN_DIM)
    b2 = b.reshape(1, NUM_EXPERTS)

    probs, logits, zloss = pl.pallas_call(
        _router_block,
        grid=(TOTAL // BLOCK_T,),
        in_specs=[
            pl.BlockSpec((BLOCK_T, HIDDEN_DIM), lambda i: (i, 0)),
            pl.BlockSpec((NUM_EXPERTS, HIDDEN_DIM), lambda i: (0, 0)),
            pl.BlockSpec((1, NUM_EXPERTS), lambda i: (0, 0)),
        ],
        out_specs=[
            pl.BlockSpec((BLOCK_T, NUM_EXPERTS), lambda i: (i, 0)),
            pl.BlockSpec((BLOCK_T, NUM_EXPERTS), lambda i: (i, 0)),
            pl.BlockSpec((1, 1), lambda i: (0, 0)),
        ],
        out_shape=[
            jax.ShapeDtypeStruct((TOTAL, NUM_EXPERTS), jnp.float32),
            jax.ShapeDtypeStruct((TOTAL, NUM_EXPERTS), jnp.float32),
            jax.ShapeDtypeStruct((1, 1), jnp.float32),
        ],
    )(x, W, b2)

    router_probs = probs.reshape(NUM_GROUPS, TOKENS_PER_GROUP, NUM_EXPERTS)
    router_logits = logits.reshape(NUM_GROUPS, TOKENS_PER_GROUP, NUM_EXPERTS)
    return (router_probs, router_logits, zloss.reshape(()))
